# baseline (speedup 1.0000x reference)
import functools

import jax
import jax.numpy as jnp
from jax import lax
from jax.experimental import pallas as pl
from jax.experimental.pallas import tpu as pltpu
from jax.experimental.pallas import tpu_sc as plsc

_D = 1024
_S = 4096
_B = 4
_NC = 2
_NS = 16
_NW = _NC * _NS
_ROWS_PER_W = _S // _NW
_CH = 64

_mesh = plsc.VectorSubcoreMesh(
    core_axis_name="c", subcore_axis_name="s", num_cores=_NC, num_subcores=_NS
)


@functools.partial(
    pl.kernel,
    mesh=_mesh,
    out_type=jax.ShapeDtypeStruct((_B, _S, _D), jnp.float32),
)
def _pos_embed_sc(table_hbm, out_hbm):
    wid = lax.axis_index("s") * _NC + lax.axis_index("c")
    base = wid * _ROWS_PER_W
    for b in range(_B):
        pltpu.sync_copy(
            table_hbm.at[pl.ds(base, _ROWS_PER_W)],
            out_hbm.at[b, pl.ds(base, _ROWS_PER_W)],
        )


def kernel(embeddings, table):
    del embeddings
    return _pos_embed_sc(table)


# speedup vs baseline: 44.3806x; 44.3806x over previous
import functools

import jax
import jax.numpy as jnp
from jax import lax
from jax.experimental import pallas as pl
from jax.experimental.pallas import tpu as pltpu
from jax.experimental.pallas import tpu_sc as plsc

_D = 1024
_S = 4096
_B = 4
_NC = 2
_NS = 16
_NW = _NC * _NS
_ROWS_PER_W = _S // _NW
_CH = 64

_mesh = plsc.VectorSubcoreMesh(
    core_axis_name="c", subcore_axis_name="s", num_cores=_NC, num_subcores=_NS
)


@functools.partial(
    pl.kernel,
    mesh=_mesh,
    out_type=jax.ShapeDtypeStruct((_B, _S, _D), jnp.float32),
    scratch_types=[
        pltpu.VMEM((_CH, _D), jnp.float32),
        pltpu.VMEM_SHARED((_NS, _CH, _D), jnp.float32),
        pltpu.SemaphoreType.DMA,
        pltpu.SemaphoreType.DMA,
        pltpu.SemaphoreType.DMA,
    ],
)
def _pos_embed_sc(table_hbm, out_hbm, buf, shared, rsem0, rsem1, wsem):
    sid = lax.axis_index("s")
    wid = sid * _NC + lax.axis_index("c")
    base = wid * _ROWS_PER_W
    r0 = pltpu.async_copy(table_hbm.at[pl.ds(base, _CH)], buf, rsem0)
    r1 = pltpu.async_copy(table_hbm.at[pl.ds(base + _CH, _CH)], shared.at[sid], rsem1)
    r0.wait()
    writes = [
        pltpu.async_copy(buf, out_hbm.at[b, pl.ds(base, _CH)], wsem)
        for b in range(_B)
    ]
    r1.wait()
    writes += [
        pltpu.async_copy(shared.at[sid], out_hbm.at[b, pl.ds(base + _CH, _CH)], wsem)
        for b in range(_B)
    ]
    for c in writes:
        c.wait()


def kernel(embeddings, table):
    del embeddings
    return _pos_embed_sc(table)


# speedup vs baseline: 44.8057x; 1.0096x over previous
import functools

import jax
import jax.numpy as jnp
from jax import lax
from jax.experimental import pallas as pl
from jax.experimental.pallas import tpu as pltpu
from jax.experimental.pallas import tpu_sc as plsc

_D = 1024
_S = 4096
_B = 4
_NC = 2
_NS = 16
_NW = _NC * _NS
_ROWS_PER_W = _S // _NW
_CH = 64

_mesh = plsc.VectorSubcoreMesh(
    core_axis_name="c", subcore_axis_name="s", num_cores=_NC, num_subcores=_NS
)


@functools.partial(
    pl.kernel,
    mesh=_mesh,
    out_type=jax.ShapeDtypeStruct((_B, _S, _D), jnp.float32),
    scratch_types=[
        pltpu.VMEM((_CH, _D), jnp.float32),
    ],
)
def _pos_embed_sc(table_hbm, out_hbm, buf):
    wid = lax.axis_index("s") * _NC + lax.axis_index("c")
    base = wid * _ROWS_PER_W
    for p in range(_ROWS_PER_W // _CH):
        off = base + p * _CH
        pltpu.sync_copy(table_hbm.at[pl.ds(off, _CH)], buf)
        for b in range(_B):
            pltpu.sync_copy(buf, out_hbm.at[b, pl.ds(off, _CH)])


def kernel(embeddings, table):
    del embeddings
    return _pos_embed_sc(table)


# speedup vs baseline: 44.8248x; 1.0004x over previous
import functools

import jax
import jax.numpy as jnp
from jax import lax
from jax.experimental import pallas as pl
from jax.experimental.pallas import tpu as pltpu
from jax.experimental.pallas import tpu_sc as plsc

_D = 1024
_S = 4096
_B = 4
_NC = 2
_NS = 16
_NW = _NC * _NS
_ROWS_PER_W = _S // _NW
_CH = 64

_mesh = plsc.VectorSubcoreMesh(
    core_axis_name="c", subcore_axis_name="s", num_cores=_NC, num_subcores=_NS
)


@functools.partial(
    pl.kernel,
    mesh=_mesh,
    out_type=jax.ShapeDtypeStruct((_B, _S, _D), jnp.float32),
    scratch_types=[
        pltpu.VMEM((_CH, _D), jnp.float32),
    ],
)
def _pos_embed_sc(table_hbm, out_hbm, buf):
    wid = lax.axis_index("s") * _NC + lax.axis_index("c")
    base = wid * _ROWS_PER_W

    def body(p, carry):
        off = base + p * _CH
        pltpu.sync_copy(table_hbm.at[pl.ds(off, _CH)], buf)
        for b in range(_B):
            pltpu.sync_copy(buf, out_hbm.at[b, pl.ds(off, _CH)])
        return carry

    lax.fori_loop(0, _ROWS_PER_W // _CH, body, 0)


def kernel(embeddings, table):
    del embeddings
    return _pos_embed_sc(table)
